# Initial kernel scaffold; baseline (speedup 1.0000x reference)
import jax
import jax.numpy as jnp
from jax.experimental import pallas as pl


def kernel(x, invariant_node_features, basis):
    raise NotImplementedError("write your pallas kernel here")


# speedup vs baseline: 94.3607x; 94.3607x over previous
import functools
import math

import jax
import jax.numpy as jnp
from jax import lax
from jax.experimental import pallas as pl
from jax.experimental.pallas import tpu as pltpu

_N_ATOMS = 100
_N_FEATURES = 64
_BATCH = 128
_BINS = 64
_STEP = 2.0 / (_BINS - 1)
_ISTEP = 1.0 / _STEP
_CKS = [(-1.0 + k * _STEP) * _ISTEP for k in range(_BINS)]
_INV2PI = 1.0 / (2.0 * math.pi)


def _asin(x):
    ax = jnp.abs(x)
    p = jnp.float32(-0.0012624911)
    for c in (0.0066700901, -0.0170881256, 0.0308918810,
              -0.0501743046, 0.0889789874, -0.2145988016, 1.5707963050):
        p = p * ax + jnp.float32(c)
    r = jnp.float32(1.5707963267948966) - jnp.sqrt(jnp.maximum(1.0 - ax, 0.0)) * p
    return jnp.sign(x) * r


def _cross(a, b):
    return (a[1] * b[2] - a[2] * b[1],
            a[2] * b[0] - a[0] * b[2],
            a[0] * b[1] - a[1] * b[0])


def _norm3(v):
    r = lax.rsqrt(v[0] * v[0] + v[1] * v[1] + v[2] * v[2])
    return (v[0] * r, v[1] * r, v[2] * r)


def _dot3(a, b):
    return a[0] * b[0] + a[1] * b[1] + a[2] * b[2]


def _shl_lane(m):
    return jnp.concatenate([m[:, 1:], m[:, :1]], axis=1)


def _shl_sub(m):
    return jnp.concatenate([m[1:, :], m[:1, :]], axis=0)


def _shr_lane_row(v):
    return jnp.concatenate([jnp.zeros((1, 1), jnp.float32), v[:, :-1]], axis=1)


def _frame_body(xt_ref, nodef_ref, basis_ref, out_ref, gt_ref):
    xr = xt_ref[0]
    rows = [jnp.broadcast_to(xr[c:c + 1, :], (128, 128)) for c in range(3)]
    cols = [jnp.transpose(r) for r in rows]
    d = [rows[c] - cols[c] for c in range(3)]
    r2 = d[0] * d[0] + d[1] * d[1] + d[2] * d[2]
    w = jnp.exp(-r2)
    inv = lax.rsqrt(r2)
    ua = tuple(d[c] * inv for c in range(3))
    ub = tuple(_shl_lane(u) for u in ua)
    uc = tuple(_shl_sub(u) for u in ua)
    ud = tuple(_shl_sub(u) for u in ub)

    c0 = _norm3(_cross(ua, ub))
    c1 = _norm3(_cross(ub, ud))
    c2 = _norm3(_cross(ud, uc))
    c3 = _norm3(_cross(uc, ua))
    omega = (_asin(jnp.clip(_dot3(c0, c1), -1.0, 1.0))
             + _asin(jnp.clip(_dot3(c1, c2), -1.0, 1.0))
             + _asin(jnp.clip(_dot3(c2, c3), -1.0, 1.0))
             + _asin(jnp.clip(_dot3(c3, c0), -1.0, 1.0)))

    tj = tuple(_shl_lane(rows[c]) - rows[c] for c in range(3))
    ti = tuple(_shl_sub(cols[c]) - cols[c] for c in range(3))
    sgn = jnp.sign(_dot3(_cross(tj, ti), ua))
    wr = omega * sgn * jnp.float32(_INV2PI)

    ii = lax.broadcasted_iota(jnp.int32, (128, 128), 0)
    jj = lax.broadcasted_iota(jnp.int32, (128, 128), 1)
    mask = (ii + 2 <= jj) & (jj <= 98)
    wrs = jnp.where(mask, wr * jnp.float32(_ISTEP), 0.0)
    w1 = jnp.where(mask, w, 0.0)
    w2 = jnp.where(mask, _shl_sub(_shl_lane(w)), 0.0)

    s1 = jnp.sum(w1, axis=0, keepdims=True)
    s2 = jnp.sum(w2, axis=0, keepdims=True)
    denom = s1 + _shr_lane_row(s2)
    dinv = jnp.where(denom > 0, jnp.float32(1.0 / 1.12) / denom, 0.0)

    for k in range(_BINS):
        dk = wrs - jnp.float32(_CKS[k])
        e = jnp.exp(-(dk * dk))
        r1 = jnp.sum(w1 * e, axis=0, keepdims=True)
        r2v = jnp.sum(w2 * e, axis=0, keepdims=True)
        gt_ref[k:k + 1, :] = r1 + _shr_lane_row(r2v)

    gt = gt_ref[...] * dinv
    msg = lax.dot_general(gt, basis_ref[...], (((0,), (0,)), ((), ())),
                          preferred_element_type=jnp.float32)
    out_ref[0] = nodef_ref[0] + msg[:_N_ATOMS, :]


@jax.jit
def kernel(x, invariant_node_features, basis):
    xt = jnp.transpose(x.reshape(_BATCH, _N_ATOMS, 3), (0, 2, 1))
    xt = jnp.pad(xt, ((0, 0), (0, 0), (0, 128 - _N_ATOMS)))
    nodef3 = invariant_node_features.reshape(_BATCH, _N_ATOMS, _N_FEATURES)
    basis2 = basis[0, 0]

    out3 = pl.pallas_call(
        _frame_body,
        grid=(_BATCH,),
        in_specs=[
            pl.BlockSpec((1, 3, 128), lambda b: (b, 0, 0)),
            pl.BlockSpec((1, _N_ATOMS, _N_FEATURES), lambda b: (b, 0, 0)),
            pl.BlockSpec((_BINS, _N_FEATURES), lambda b: (0, 0)),
        ],
        out_specs=pl.BlockSpec((1, _N_ATOMS, _N_FEATURES), lambda b: (b, 0, 0)),
        out_shape=jax.ShapeDtypeStruct((_BATCH, _N_ATOMS, _N_FEATURES), jnp.float32),
        scratch_shapes=[pltpu.VMEM((_BINS, 128), jnp.float32)],
    )(xt, nodef3, basis2)
    return out3.reshape(_BATCH * _N_ATOMS, _N_FEATURES)
